# baseline (device time: 169342 ns/iter reference)
import functools

import jax
import jax.numpy as jnp
from jax import lax
from jax.experimental import pallas as pl
from jax.experimental.pallas import tpu as pltpu

N_DEV = 8
SQ = 2048
SKV = 2048
D_MODEL = 1024
H_PER = 8
DH = 128
SCALE = 0.08838834764831843
CHUNK = SQ // N_DEV


QT = 256
N_QT = SQ // QT
KT = 256
N_KT = SKV // KT


def _key_spans(qt: int) -> list[tuple[int, int]]:
    if qt == 0:
        return [(0, SKV)]
    band = [(t * KT, KT) for t in (qt - 1, qt, qt + 1) if 0 <= t < N_KT]
    if qt - 1 > 0:
        band = [(0, 128)] + band
    return band


def _attn_body(x_ref, wq_ref, k_ref, v_ref, ctx_ref):
    xm = x_ref[0].astype(jnp.bfloat16)
    q = jnp.dot(
        xm, wq_ref[...].astype(jnp.bfloat16),
        preferred_element_type=jnp.float32,
    )
    k = k_ref[...].astype(jnp.bfloat16)
    v = v_ref[...].astype(jnp.bfloat16)
    for qt in range(N_QT):
        spans = _key_spans(qt)
        q_t = q[qt * QT:(qt + 1) * QT, :].astype(jnp.bfloat16)
        k_sel = jnp.concatenate([k[o:o + w_, :] for o, w_ in spans], 0)
        v_sel = jnp.concatenate([v[o:o + w_, :] for o, w_ in spans], 0)
        s = lax.dot_general(
            q_t, k_sel, (((1,), (1,)), ((), ())),
            preferred_element_type=jnp.float32,
        ) * SCALE
        qi = qt * QT + lax.broadcasted_iota(jnp.int32, s.shape, 0)
        ki = jnp.concatenate(
            [o + lax.broadcasted_iota(jnp.int32, (QT, w_), 1) for o, w_ in spans],
            axis=1,
        )
        mask = (jnp.abs(qi - ki) <= 128) | (ki < 32) | (qi < 32)
        s = jnp.where(mask, s, -1e9)
        m = jnp.max(s, axis=1, keepdims=True)
        w = jnp.exp(s - m)
        w = w / jnp.sum(w, axis=1, keepdims=True)
        ctx_ref[qt * QT:(qt + 1) * QT, :] = jnp.dot(
            w.astype(jnp.bfloat16), v_sel, preferred_element_type=jnp.float32
        )


def _attention(x, Wq_l, K_flat, V_flat):
    return pl.pallas_call(
        _attn_body,
        grid=(H_PER,),
        in_specs=[
            pl.BlockSpec((1, SQ, D_MODEL), lambda h: (0, 0, 0)),
            pl.BlockSpec((D_MODEL, DH), lambda h: (0, h)),
            pl.BlockSpec((SKV, DH), lambda h: (0, h)),
            pl.BlockSpec((SKV, DH), lambda h: (0, h)),
        ],
        out_specs=pl.BlockSpec((SQ, DH), lambda h: (0, h)),
        out_shape=jax.ShapeDtypeStruct((SQ, H_PER * DH), jnp.float32),
    )(x, Wq_l, K_flat, V_flat)


HCHUNK = SQ // 2 // N_DEV


def _allreduce_body(ctx_ref, wo_ref, out_ref, send_buf, recv_buf, sems):
    my = lax.axis_index("i")
    left = lax.rem(my + N_DEV - 1, N_DEV)
    right = lax.rem(my + 1, N_DEV)

    def rows_of(ring, c):
        return pl.ds(ring * (SQ // 2) + c * HCHUNK, HCHUNK)

    def project(ring, c):
        r = rows_of(ring, c)
        out_ref[0, r, :] = jnp.dot(
            ctx_ref[r, :], wo_ref[...], preferred_element_type=jnp.float32
        )

    project(0, my)
    project(1, my)

    barrier_sem = pltpu.get_barrier_semaphore()
    for nbr in (left, right):
        pl.semaphore_signal(
            barrier_sem, inc=1,
            device_id=(nbr,), device_id_type=pl.DeviceIdType.MESH,
        )
    pl.semaphore_wait(barrier_sem, 2)

    def hop(ring, phase, s, send_c, recv_c, accumulate):
        k = ring * 2 + phase
        send_buf[k, s] = out_ref[0, rows_of(ring, send_c), :].astype(jnp.bfloat16)
        rdma = pltpu.make_async_remote_copy(
            src_ref=send_buf.at[k, s],
            dst_ref=recv_buf.at[k, s],
            send_sem=sems.at[0, k, s],
            recv_sem=sems.at[1, k, s],
            device_id=(right if ring == 0 else left,),
            device_id_type=pl.DeviceIdType.MESH,
        )
        rdma.start()
        return rdma, (ring, recv_c, accumulate, k, s)

    def finish(pending):
        rdma, (ring, recv_c, accumulate, k, s) = pending
        rdma.wait()
        got = recv_buf[k, s].astype(jnp.float32)
        if accumulate:
            out_ref[0, rows_of(ring, recv_c), :] += got
        else:
            out_ref[0, rows_of(ring, recv_c), :] = got

    for s in range(N_DEV - 1):
        rc_a = lax.rem(my - s - 1 + N_DEV, N_DEV)
        rc_b = lax.rem(my + s + 1, N_DEV)
        pa = hop(0, 0, s, lax.rem(my - s + N_DEV, N_DEV), rc_a, True)
        pb = hop(1, 0, s, lax.rem(my + s, N_DEV), rc_b, True)
        project(0, rc_a)
        project(1, rc_b)
        finish(pa)
        finish(pb)

    for t in range(N_DEV - 1):
        pa = hop(0, 1, t, lax.rem(my + 1 - t + N_DEV, N_DEV),
                 lax.rem(my - t + N_DEV, N_DEV), False)
        pb = hop(1, 1, t, lax.rem(my - 1 + t + N_DEV, N_DEV),
                 lax.rem(my + t, N_DEV), False)
        finish(pa)
        finish(pb)


def _project_allreduce(ctx, Wo_l):
    return pl.pallas_call(
        _allreduce_body,
        out_shape=jax.ShapeDtypeStruct((1, SQ, D_MODEL), jnp.float32),
        in_specs=[
            pl.BlockSpec(memory_space=pltpu.VMEM),
            pl.BlockSpec(memory_space=pltpu.VMEM),
        ],
        out_specs=pl.BlockSpec(memory_space=pltpu.VMEM),
        scratch_shapes=[
            pltpu.VMEM((4, N_DEV - 1, HCHUNK, D_MODEL), jnp.bfloat16),
            pltpu.VMEM((4, N_DEV - 1, HCHUNK, D_MODEL), jnp.bfloat16),
            pltpu.SemaphoreType.DMA((2, 4, N_DEV - 1)),
        ],
        compiler_params=pltpu.CompilerParams(collective_id=0),
    )(ctx, Wo_l)


def kernel(x, Wq, K_ext, V_ext, Wo):
    idx = lax.axis_index("i")
    Wq_l = lax.dynamic_slice(Wq, (0, idx * H_PER * DH), (D_MODEL, H_PER * DH))
    Wo_l = lax.dynamic_slice(Wo, (idx * H_PER * DH, 0), (H_PER * DH, D_MODEL))
    K = K_ext.reshape(SKV, H_PER * DH)
    V = V_ext.reshape(SKV, H_PER * DH)
    ctx = _attention(x, Wq_l, K, V)
    return _project_allreduce(ctx, Wo_l)


# device time: 166302 ns/iter; 1.0183x vs baseline; 1.0183x over previous
import functools

import jax
import jax.numpy as jnp
from jax import lax
from jax.experimental import pallas as pl
from jax.experimental.pallas import tpu as pltpu

N_DEV = 8
SQ = 2048
SKV = 2048
D_MODEL = 1024
H_PER = 8
DH = 128
SCALE = 0.08838834764831843
CHUNK = SQ // N_DEV


QT = 256
N_QT = SQ // QT
KT = 256
N_KT = SKV // KT


def _key_spans(qt: int) -> list[tuple[int, int]]:
    if qt == 0:
        return [(0, SKV)]
    band = [(t * KT, KT) for t in (qt - 1, qt, qt + 1) if 0 <= t < N_KT]
    if qt - 1 > 0:
        band = [(0, 128)] + band
    return band


def _attn_body(x_ref, wq_ref, k_ref, v_ref, ctx_ref):
    xm = x_ref[0]
    q = jnp.dot(xm, wq_ref[...], preferred_element_type=jnp.float32)
    k = k_ref[...]
    v = v_ref[...]
    for qt in range(N_QT):
        spans = _key_spans(qt)
        q_t = q[qt * QT:(qt + 1) * QT, :]
        k_sel = jnp.concatenate([k[o:o + w_, :] for o, w_ in spans], 0)
        v_sel = jnp.concatenate([v[o:o + w_, :] for o, w_ in spans], 0)
        s = lax.dot_general(
            q_t, k_sel, (((1,), (1,)), ((), ())),
            preferred_element_type=jnp.float32,
        ) * SCALE
        qi = qt * QT + lax.broadcasted_iota(jnp.int32, s.shape, 0)
        ki = jnp.concatenate(
            [o + lax.broadcasted_iota(jnp.int32, (QT, w_), 1) for o, w_ in spans],
            axis=1,
        )
        mask = (jnp.abs(qi - ki) <= 128) | (ki < 32) | (qi < 32)
        s = jnp.where(mask, s, -1e9)
        m = jnp.max(s, axis=1, keepdims=True)
        w = jnp.exp(s - m)
        w = w / jnp.sum(w, axis=1, keepdims=True)
        ctx_ref[qt * QT:(qt + 1) * QT, :] = jnp.dot(
            w, v_sel, preferred_element_type=jnp.float32
        )


def _attention(x, Wq_l, K_flat, V_flat):
    return pl.pallas_call(
        _attn_body,
        grid=(H_PER,),
        in_specs=[
            pl.BlockSpec((1, SQ, D_MODEL), lambda h: (0, 0, 0)),
            pl.BlockSpec((D_MODEL, DH), lambda h: (0, h)),
            pl.BlockSpec((SKV, DH), lambda h: (0, h)),
            pl.BlockSpec((SKV, DH), lambda h: (0, h)),
        ],
        out_specs=pl.BlockSpec((SQ, DH), lambda h: (0, h)),
        out_shape=jax.ShapeDtypeStruct((SQ, H_PER * DH), jnp.float32),
    )(x, Wq_l, K_flat, V_flat)


HCHUNK = SQ // 2 // N_DEV


def _allreduce_body(ctx_ref, wo_ref, out_ref, send_buf, recv_buf, sems):
    my = lax.axis_index("i")
    left = lax.rem(my + N_DEV - 1, N_DEV)
    right = lax.rem(my + 1, N_DEV)

    def rows_of(ring, c):
        return pl.ds(ring * (SQ // 2) + c * HCHUNK, HCHUNK)

    def project(ring, c):
        r = rows_of(ring, c)
        out_ref[0, r, :] = jnp.dot(
            ctx_ref[r, :], wo_ref[...], preferred_element_type=jnp.float32
        )

    project(0, my)
    project(1, my)

    barrier_sem = pltpu.get_barrier_semaphore()
    for nbr in (left, right):
        pl.semaphore_signal(
            barrier_sem, inc=1,
            device_id=(nbr,), device_id_type=pl.DeviceIdType.MESH,
        )
    pl.semaphore_wait(barrier_sem, 2)

    def hop(ring, phase, s, send_c, recv_c, accumulate):
        k = ring * 2 + phase
        send_buf[k, s] = out_ref[0, rows_of(ring, send_c), :].astype(jnp.bfloat16)
        rdma = pltpu.make_async_remote_copy(
            src_ref=send_buf.at[k, s],
            dst_ref=recv_buf.at[k, s],
            send_sem=sems.at[0, k, s],
            recv_sem=sems.at[1, k, s],
            device_id=(right if ring == 0 else left,),
            device_id_type=pl.DeviceIdType.MESH,
        )
        rdma.start()
        return rdma, (ring, recv_c, accumulate, k, s)

    def finish(pending):
        rdma, (ring, recv_c, accumulate, k, s) = pending
        rdma.wait()
        got = recv_buf[k, s].astype(jnp.float32)
        if accumulate:
            out_ref[0, rows_of(ring, recv_c), :] += got
        else:
            out_ref[0, rows_of(ring, recv_c), :] = got

    for s in range(N_DEV - 1):
        rc_a = lax.rem(my - s - 1 + N_DEV, N_DEV)
        rc_b = lax.rem(my + s + 1, N_DEV)
        pa = hop(0, 0, s, lax.rem(my - s + N_DEV, N_DEV), rc_a, True)
        pb = hop(1, 0, s, lax.rem(my + s, N_DEV), rc_b, True)
        project(0, rc_a)
        project(1, rc_b)
        finish(pa)
        finish(pb)

    for t in range(N_DEV - 1):
        pa = hop(0, 1, t, lax.rem(my + 1 - t + N_DEV, N_DEV),
                 lax.rem(my - t + N_DEV, N_DEV), False)
        pb = hop(1, 1, t, lax.rem(my - 1 + t + N_DEV, N_DEV),
                 lax.rem(my + t, N_DEV), False)
        finish(pa)
        finish(pb)


def _project_allreduce(ctx, Wo_l):
    return pl.pallas_call(
        _allreduce_body,
        out_shape=jax.ShapeDtypeStruct((1, SQ, D_MODEL), jnp.float32),
        in_specs=[
            pl.BlockSpec(memory_space=pltpu.VMEM),
            pl.BlockSpec(memory_space=pltpu.VMEM),
        ],
        out_specs=pl.BlockSpec(memory_space=pltpu.VMEM),
        scratch_shapes=[
            pltpu.VMEM((4, N_DEV - 1, HCHUNK, D_MODEL), jnp.bfloat16),
            pltpu.VMEM((4, N_DEV - 1, HCHUNK, D_MODEL), jnp.bfloat16),
            pltpu.SemaphoreType.DMA((2, 4, N_DEV - 1)),
        ],
        compiler_params=pltpu.CompilerParams(collective_id=0),
    )(ctx, Wo_l)


def kernel(x, Wq, K_ext, V_ext, Wo):
    idx = lax.axis_index("i")
    Wq_l = lax.dynamic_slice(Wq, (0, idx * H_PER * DH), (D_MODEL, H_PER * DH))
    Wo_l = lax.dynamic_slice(Wo, (idx * H_PER * DH, 0), (H_PER * DH, D_MODEL))
    K = K_ext.reshape(SKV, H_PER * DH)
    V = V_ext.reshape(SKV, H_PER * DH)
    ctx = _attention(x, Wq_l, K, V)
    return _project_allreduce(ctx, Wo_l)


# device time: 156896 ns/iter; 1.0793x vs baseline; 1.0600x over previous
import functools

import jax
import jax.numpy as jnp
from jax import lax
from jax.experimental import pallas as pl
from jax.experimental.pallas import tpu as pltpu

N_DEV = 8
SQ = 2048
SKV = 2048
D_MODEL = 1024
H_PER = 8
DH = 128
SCALE = 0.08838834764831843
CHUNK = SQ // N_DEV


QT = 256
N_QT = SQ // QT
KT = 256
N_KT = SKV // KT


def _key_spans(qt: int) -> list[tuple[int, int]]:
    if qt == 0:
        return [(0, SKV)]
    band = [(t * KT, KT) for t in (qt - 1, qt, qt + 1) if 0 <= t < N_KT]
    if qt - 1 > 0:
        band = [(0, KT)] + band
    return band


def _attn_body(x_ref, wq_ref, k_ref, v_ref, ctx_ref):
    xm = x_ref[0]
    q = jnp.dot(xm, wq_ref[...], preferred_element_type=jnp.float32)
    k = k_ref[...]
    v = v_ref[...]
    for qt in range(N_QT):
        spans = _key_spans(qt)
        q_t = q[qt * QT:(qt + 1) * QT, :]
        k_sel = jnp.concatenate([k[o:o + w_, :] for o, w_ in spans], 0)
        v_sel = jnp.concatenate([v[o:o + w_, :] for o, w_ in spans], 0)
        s = lax.dot_general(
            q_t, k_sel, (((1,), (1,)), ((), ())),
            preferred_element_type=jnp.float32,
        ) * SCALE
        qi = qt * QT + lax.broadcasted_iota(jnp.int32, s.shape, 0)
        ki = jnp.concatenate(
            [o + lax.broadcasted_iota(jnp.int32, (QT, w_), 1) for o, w_ in spans],
            axis=1,
        )
        mask = (jnp.abs(qi - ki) <= 128) | (ki < 32) | (qi < 32)
        s = jnp.where(mask, s, -1e9)
        m = jnp.max(s, axis=1, keepdims=True)
        w = jnp.exp(s - m)
        w = w / jnp.sum(w, axis=1, keepdims=True)
        ctx_ref[qt * QT:(qt + 1) * QT, :] = jnp.dot(
            w, v_sel, preferred_element_type=jnp.float32
        )


def _attention(x, Wq_l, K_flat, V_flat):
    return pl.pallas_call(
        _attn_body,
        grid=(H_PER,),
        in_specs=[
            pl.BlockSpec((1, SQ, D_MODEL), lambda h: (0, 0, 0)),
            pl.BlockSpec((D_MODEL, DH), lambda h: (0, h)),
            pl.BlockSpec((SKV, DH), lambda h: (0, h)),
            pl.BlockSpec((SKV, DH), lambda h: (0, h)),
        ],
        out_specs=pl.BlockSpec((SQ, DH), lambda h: (0, h)),
        out_shape=jax.ShapeDtypeStruct((SQ, H_PER * DH), jnp.float32),
    )(x, Wq_l, K_flat, V_flat)


HCHUNK = SQ // 2 // N_DEV


def _allreduce_body(ctx_ref, wo_ref, out_ref, send_buf, recv_buf, sems):
    my = lax.axis_index("i")
    left = lax.rem(my + N_DEV - 1, N_DEV)
    right = lax.rem(my + 1, N_DEV)

    def rows_of(ring, c):
        return pl.ds(ring * (SQ // 2) + c * HCHUNK, HCHUNK)

    def project(ring, c):
        r = rows_of(ring, c)
        out_ref[0, r, :] = jnp.dot(
            ctx_ref[r, :], wo_ref[...], preferred_element_type=jnp.float32
        )

    project(0, my)
    project(1, my)

    barrier_sem = pltpu.get_barrier_semaphore()
    for nbr in (left, right):
        pl.semaphore_signal(
            barrier_sem, inc=1,
            device_id=(nbr,), device_id_type=pl.DeviceIdType.MESH,
        )
    pl.semaphore_wait(barrier_sem, 2)

    def hop(ring, phase, s, send_c, recv_c, accumulate):
        k = ring * 2 + phase
        send_buf[k, s] = out_ref[0, rows_of(ring, send_c), :].astype(jnp.bfloat16)
        rdma = pltpu.make_async_remote_copy(
            src_ref=send_buf.at[k, s],
            dst_ref=recv_buf.at[k, s],
            send_sem=sems.at[0, k, s],
            recv_sem=sems.at[1, k, s],
            device_id=(right if ring == 0 else left,),
            device_id_type=pl.DeviceIdType.MESH,
        )
        rdma.start()
        return rdma, (ring, recv_c, accumulate, k, s)

    def finish(pending):
        rdma, (ring, recv_c, accumulate, k, s) = pending
        rdma.wait()
        got = recv_buf[k, s].astype(jnp.float32)
        if accumulate:
            out_ref[0, rows_of(ring, recv_c), :] += got
        else:
            out_ref[0, rows_of(ring, recv_c), :] = got

    for s in range(N_DEV - 1):
        rc_a = lax.rem(my - s - 1 + N_DEV, N_DEV)
        rc_b = lax.rem(my + s + 1, N_DEV)
        pa = hop(0, 0, s, lax.rem(my - s + N_DEV, N_DEV), rc_a, True)
        pb = hop(1, 0, s, lax.rem(my + s, N_DEV), rc_b, True)
        project(0, rc_a)
        project(1, rc_b)
        finish(pa)
        finish(pb)

    for t in range(N_DEV - 1):
        pa = hop(0, 1, t, lax.rem(my + 1 - t + N_DEV, N_DEV),
                 lax.rem(my - t + N_DEV, N_DEV), False)
        pb = hop(1, 1, t, lax.rem(my - 1 + t + N_DEV, N_DEV),
                 lax.rem(my + t, N_DEV), False)
        finish(pa)
        finish(pb)


def _project_allreduce(ctx, Wo_l):
    return pl.pallas_call(
        _allreduce_body,
        out_shape=jax.ShapeDtypeStruct((1, SQ, D_MODEL), jnp.float32),
        in_specs=[
            pl.BlockSpec(memory_space=pltpu.VMEM),
            pl.BlockSpec(memory_space=pltpu.VMEM),
        ],
        out_specs=pl.BlockSpec(memory_space=pltpu.VMEM),
        scratch_shapes=[
            pltpu.VMEM((4, N_DEV - 1, HCHUNK, D_MODEL), jnp.bfloat16),
            pltpu.VMEM((4, N_DEV - 1, HCHUNK, D_MODEL), jnp.bfloat16),
            pltpu.SemaphoreType.DMA((2, 4, N_DEV - 1)),
        ],
        compiler_params=pltpu.CompilerParams(collective_id=0),
    )(ctx, Wo_l)


def kernel(x, Wq, K_ext, V_ext, Wo):
    idx = lax.axis_index("i")
    Wq_l = lax.dynamic_slice(Wq, (0, idx * H_PER * DH), (D_MODEL, H_PER * DH))
    Wo_l = lax.dynamic_slice(Wo, (idx * H_PER * DH, 0), (H_PER * DH, D_MODEL))
    K = K_ext.reshape(SKV, H_PER * DH)
    V = V_ext.reshape(SKV, H_PER * DH)
    ctx = _attention(x, Wq_l, K, V)
    return _project_allreduce(ctx, Wo_l)
